# initial kernel scaffold (unmeasured)
import jax
import jax.numpy as jnp
from jax import lax
from jax.experimental import pallas as pl
from jax.experimental.pallas import tpu as pltpu

N_DEV = 8


def kernel(x, w_mat):
    m, k = x.shape
    _, n = w_mat.shape
    chunk = m // N_DEV

    def body(x_ref, w_ref, out_ref, comm_ref, rs_send, rs_recv, ag_send, ag_recv):
        my = lax.axis_index("i")
        left = lax.rem(my + (N_DEV - 1), N_DEV)
        right = lax.rem(my + 1, N_DEV)

        barrier_sem = pltpu.get_barrier_semaphore()
        for nbr in (left, right):
            pl.semaphore_signal(
                barrier_sem, inc=1,
                device_id=(nbr,), device_id_type=pl.DeviceIdType.MESH,
            )
        pl.semaphore_wait(barrier_sem, 2)

        out_ref[...] = jnp.dot(
            x_ref[...], w_ref[...], preferred_element_type=jnp.float32
        )

        for h in range(N_DEV - 1):
            sc = lax.rem(my - h + N_DEV, N_DEV)
            rdma = pltpu.make_async_remote_copy(
                src_ref=out_ref.at[pl.ds(sc * chunk, chunk), :],
                dst_ref=comm_ref.at[h],
                send_sem=rs_send.at[h],
                recv_sem=rs_recv.at[h],
                device_id=(right,),
                device_id_type=pl.DeviceIdType.MESH,
            )
            rdma.start()
            rdma.wait()
            rc = lax.rem(my - h - 1 + N_DEV, N_DEV)
            out_ref[pl.ds(rc * chunk, chunk), :] += comm_ref[h]

        own = lax.rem(my + 1, N_DEV)
        y = out_ref[pl.ds(own * chunk, chunk), :]
        out_ref[pl.ds(own * chunk, chunk), :] = y * jax.nn.sigmoid(y)

        for h in range(N_DEV - 1):
            sc = lax.rem(my + 1 - h + N_DEV, N_DEV)
            rdma = pltpu.make_async_remote_copy(
                src_ref=out_ref.at[pl.ds(sc * chunk, chunk), :],
                dst_ref=out_ref.at[pl.ds(sc * chunk, chunk), :],
                send_sem=ag_send.at[h],
                recv_sem=ag_recv.at[h],
                device_id=(right,),
                device_id_type=pl.DeviceIdType.MESH,
            )
            rdma.start()
            rdma.wait()

    return pl.pallas_call(
        body,
        out_shape=jax.ShapeDtypeStruct((m, n), jnp.float32),
        in_specs=[
            pl.BlockSpec(memory_space=pltpu.VMEM),
            pl.BlockSpec(memory_space=pltpu.VMEM),
        ],
        out_specs=pl.BlockSpec(memory_space=pltpu.VMEM),
        scratch_shapes=[
            pltpu.VMEM((N_DEV - 1, chunk, n), jnp.float32),
            pltpu.SemaphoreType.DMA((N_DEV - 1,)),
            pltpu.SemaphoreType.DMA((N_DEV - 1,)),
            pltpu.SemaphoreType.DMA((N_DEV - 1,)),
            pltpu.SemaphoreType.DMA((N_DEV - 1,)),
        ],
        compiler_params=pltpu.CompilerParams(collective_id=0),
    )(x, w_mat)


# baseline (device time: 561495 ns/iter reference)
import jax
import jax.numpy as jnp
from jax import lax
from jax.experimental import pallas as pl
from jax.experimental.pallas import tpu as pltpu

N_DEV = 8


def kernel(x, w_mat):
    m, k = x.shape
    _, n = w_mat.shape
    chunk = m // N_DEV

    x = x.astype(jnp.bfloat16)
    w_mat = w_mat.astype(jnp.bfloat16)

    def body(x_ref, w_ref, out_ref, rsbuf, stage, rs_send, rs_recv, ag_send, ag_recv):
        my = lax.axis_index("i")
        left = lax.rem(my + (N_DEV - 1), N_DEV)
        right = lax.rem(my + 1, N_DEV)

        barrier_sem = pltpu.get_barrier_semaphore()
        for nbr in (left, right):
            pl.semaphore_signal(
                barrier_sem, inc=1,
                device_id=(nbr,), device_id_type=pl.DeviceIdType.MESH,
            )
        pl.semaphore_wait(barrier_sem, 2)

        out_ref[...] = jnp.dot(
            x_ref[...], w_ref[...], preferred_element_type=jnp.float32
        )

        for h in range(N_DEV - 1):
            sc = lax.rem(my - h + N_DEV, N_DEV)
            slot = h % 2
            stage[slot] = out_ref[pl.ds(sc * chunk, chunk), :].astype(jnp.bfloat16)
            rdma = pltpu.make_async_remote_copy(
                src_ref=stage.at[slot],
                dst_ref=rsbuf.at[h],
                send_sem=rs_send.at[h],
                recv_sem=rs_recv.at[h],
                device_id=(right,),
                device_id_type=pl.DeviceIdType.MESH,
            )
            rdma.start()
            rdma.wait()
            rc = lax.rem(my - h - 1 + N_DEV, N_DEV)
            out_ref[pl.ds(rc * chunk, chunk), :] += rsbuf[h].astype(jnp.float32)

        own = lax.rem(my + 1, N_DEV)
        y = out_ref[pl.ds(own * chunk, chunk), :]
        out_ref[pl.ds(own * chunk, chunk), :] = y * jax.nn.sigmoid(y)

        for h in range(N_DEV - 1):
            sc = lax.rem(my + 1 - h + N_DEV, N_DEV)
            rdma = pltpu.make_async_remote_copy(
                src_ref=out_ref.at[pl.ds(sc * chunk, chunk), :],
                dst_ref=out_ref.at[pl.ds(sc * chunk, chunk), :],
                send_sem=ag_send.at[h],
                recv_sem=ag_recv.at[h],
                device_id=(right,),
                device_id_type=pl.DeviceIdType.MESH,
            )
            rdma.start()
            rdma.wait()

    return pl.pallas_call(
        body,
        out_shape=jax.ShapeDtypeStruct((m, n), jnp.float32),
        in_specs=[
            pl.BlockSpec(memory_space=pltpu.VMEM),
            pl.BlockSpec(memory_space=pltpu.VMEM),
        ],
        out_specs=pl.BlockSpec(memory_space=pltpu.VMEM),
        scratch_shapes=[
            pltpu.VMEM((N_DEV - 1, chunk, n), jnp.bfloat16),
            pltpu.VMEM((2, chunk, n), jnp.bfloat16),
            pltpu.SemaphoreType.DMA((N_DEV - 1,)),
            pltpu.SemaphoreType.DMA((N_DEV - 1,)),
            pltpu.SemaphoreType.DMA((N_DEV - 1,)),
            pltpu.SemaphoreType.DMA((N_DEV - 1,)),
        ],
        compiler_params=pltpu.CompilerParams(
            collective_id=0,
            vmem_limit_bytes=100 * 1024 * 1024,
        ),
    )(x, w_mat)


# device time: 255377 ns/iter; 2.1987x vs baseline; 2.1987x over previous
import jax
import jax.numpy as jnp
from jax import lax
from jax.experimental import pallas as pl
from jax.experimental.pallas import tpu as pltpu

N_DEV = 8
N_HOP = N_DEV - 1


def kernel(x, w_mat):
    m, k = x.shape
    _, n = w_mat.shape
    chunk = m // N_DEV
    half = n // 2

    x = x.astype(jnp.bfloat16)
    w_mat = w_mat.astype(jnp.bfloat16)

    def body(x_ref, w_ref, out_ref, rsbuf, stage,
             send_p, recv_p, send_m, recv_m,
             ag_send_p, ag_recv_p, ag_send_m, ag_recv_m,
             phase_sem):
        my = lax.axis_index("i")
        left = lax.rem(my + (N_DEV - 1), N_DEV)
        right = lax.rem(my + 1, N_DEV)

        barrier_sem = pltpu.get_barrier_semaphore()
        for nbr in (left, right):
            pl.semaphore_signal(
                barrier_sem, inc=1,
                device_id=(nbr,), device_id_type=pl.DeviceIdType.MESH,
            )
        pl.semaphore_wait(barrier_sem, 2)

        out_ref[...] = jnp.dot(
            x_ref[...], w_ref[...], preferred_element_type=jnp.float32
        )

        def rows(c):
            return pl.ds(c * chunk, chunk)

        for h in range(N_HOP):
            slot = h % 2
            sp = lax.rem(my - h + N_DEV, N_DEV)
            sm = lax.rem(my + h, N_DEV)
            stage[0, slot] = out_ref[rows(sp), :half].astype(jnp.bfloat16)
            stage[1, slot] = out_ref[rows(sm), half:].astype(jnp.bfloat16)
            rdma_p = pltpu.make_async_remote_copy(
                src_ref=stage.at[0, slot], dst_ref=rsbuf.at[0, h],
                send_sem=send_p.at[h], recv_sem=recv_p.at[h],
                device_id=(right,), device_id_type=pl.DeviceIdType.MESH,
            )
            rdma_m = pltpu.make_async_remote_copy(
                src_ref=stage.at[1, slot], dst_ref=rsbuf.at[1, h],
                send_sem=send_m.at[h], recv_sem=recv_m.at[h],
                device_id=(left,), device_id_type=pl.DeviceIdType.MESH,
            )
            rdma_p.start()
            rdma_m.start()
            rdma_p.wait()
            rdma_m.wait()
            rp = lax.rem(my - h - 1 + N_DEV, N_DEV)
            rm = lax.rem(my + h + 1, N_DEV)
            out_ref[rows(rp), :half] += rsbuf[0, h].astype(jnp.float32)
            out_ref[rows(rm), half:] += rsbuf[1, h].astype(jnp.float32)

        own_p = lax.rem(my + 1, N_DEV)
        own_m = lax.rem(my + (N_DEV - 1), N_DEV)
        yp = out_ref[rows(own_p), :half]
        ym = out_ref[rows(own_m), half:]
        yp = yp * jax.nn.sigmoid(yp)
        ym = ym * jax.nn.sigmoid(ym)
        out_ref[rows(own_p), :half] = yp
        out_ref[rows(own_m), half:] = ym
        stage[0, 0] = yp.astype(jnp.bfloat16)
        stage[1, 0] = ym.astype(jnp.bfloat16)

        pl.semaphore_signal(phase_sem, inc=1, device_id=(left,),
                            device_id_type=pl.DeviceIdType.MESH)
        pl.semaphore_signal(phase_sem, inc=1, device_id=(right,),
                            device_id_type=pl.DeviceIdType.MESH)
        pl.semaphore_wait(phase_sem, 2)

        for h in range(N_HOP):
            src_p = stage.at[0, 0] if h == 0 else rsbuf.at[0, h - 1]
            src_m = stage.at[1, 0] if h == 0 else rsbuf.at[1, h - 1]
            rdma_p = pltpu.make_async_remote_copy(
                src_ref=src_p, dst_ref=rsbuf.at[0, h],
                send_sem=ag_send_p.at[h], recv_sem=ag_recv_p.at[h],
                device_id=(right,), device_id_type=pl.DeviceIdType.MESH,
            )
            rdma_m = pltpu.make_async_remote_copy(
                src_ref=src_m, dst_ref=rsbuf.at[1, h],
                send_sem=ag_send_m.at[h], recv_sem=ag_recv_m.at[h],
                device_id=(left,), device_id_type=pl.DeviceIdType.MESH,
            )
            rdma_p.start()
            rdma_m.start()
            if h > 0:
                rp_prev = lax.rem(my - (h - 1) + N_DEV, N_DEV)
                rm_prev = lax.rem(my + (h - 1), N_DEV)
                out_ref[rows(rp_prev), :half] = rsbuf[0, h - 1].astype(jnp.float32)
                out_ref[rows(rm_prev), half:] = rsbuf[1, h - 1].astype(jnp.float32)
            rdma_p.wait()
            rdma_m.wait()
        rp_last = lax.rem(my - (N_HOP - 1) + N_DEV, N_DEV)
        rm_last = lax.rem(my + (N_HOP - 1), N_DEV)
        out_ref[rows(rp_last), :half] = rsbuf[0, N_HOP - 1].astype(jnp.float32)
        out_ref[rows(rm_last), half:] = rsbuf[1, N_HOP - 1].astype(jnp.float32)

    return pl.pallas_call(
        body,
        out_shape=jax.ShapeDtypeStruct((m, n), jnp.float32),
        in_specs=[
            pl.BlockSpec(memory_space=pltpu.VMEM),
            pl.BlockSpec(memory_space=pltpu.VMEM),
        ],
        out_specs=pl.BlockSpec(memory_space=pltpu.VMEM),
        scratch_shapes=[
            pltpu.VMEM((2, N_HOP, chunk, half), jnp.bfloat16),
            pltpu.VMEM((2, 2, chunk, half), jnp.bfloat16),
            pltpu.SemaphoreType.DMA((N_HOP,)),
            pltpu.SemaphoreType.DMA((N_HOP,)),
            pltpu.SemaphoreType.DMA((N_HOP,)),
            pltpu.SemaphoreType.DMA((N_HOP,)),
            pltpu.SemaphoreType.DMA((N_HOP,)),
            pltpu.SemaphoreType.DMA((N_HOP,)),
            pltpu.SemaphoreType.DMA((N_HOP,)),
            pltpu.SemaphoreType.DMA((N_HOP,)),
            pltpu.SemaphoreType.REGULAR,
        ],
        compiler_params=pltpu.CompilerParams(
            collective_id=0,
            vmem_limit_bytes=100 * 1024 * 1024,
        ),
    )(x, w_mat)


# device time: 230442 ns/iter; 2.4366x vs baseline; 1.1082x over previous
import jax
import jax.numpy as jnp
from jax import lax
from jax.experimental import pallas as pl
from jax.experimental.pallas import tpu as pltpu

N_DEV = 8
N_HOP = N_DEV - 1


def kernel(x, w_mat):
    m, k = x.shape
    _, n = w_mat.shape
    chunk = m // N_DEV
    quarter = n // 4

    x = x.astype(jnp.bfloat16)
    w_mat = w_mat.astype(jnp.bfloat16)

    streams = [(0, 0), (0, 1), (1, 0), (1, 1)]

    def col(d, q):
        c0 = (2 * d + q) * quarter
        return slice(c0, c0 + quarter)

    def body(x_ref, w_ref, out_ref, rsbuf, stage,
             rs_send, rs_recv, ag_send, ag_recv, phase_sem):
        my = lax.axis_index("i")
        left = lax.rem(my + (N_DEV - 1), N_DEV)
        right = lax.rem(my + 1, N_DEV)

        def rows(c):
            return pl.ds(c * chunk, chunk)

        def chunk_id(d, h):
            if d == 0:
                return lax.rem(my - h - 1 + 2 * N_DEV, N_DEV)
            return lax.rem(my + h + 1, N_DEV)

        def gemm(c):
            out_ref[rows(c), :] = jnp.dot(
                x_ref[rows(c), :], w_ref[...],
                preferred_element_type=jnp.float32,
            )

        def rs_rdma(d, q, h, src):
            return pltpu.make_async_remote_copy(
                src_ref=src,
                dst_ref=rsbuf.at[d, q, h],
                send_sem=rs_send.at[d, q, h],
                recv_sem=rs_recv.at[d, q, h],
                device_id=(right if d == 0 else left,),
                device_id_type=pl.DeviceIdType.MESH,
            )

        def ag_rdma(d, q, h, src):
            return pltpu.make_async_remote_copy(
                src_ref=src,
                dst_ref=rsbuf.at[d, q, h],
                send_sem=ag_send.at[d, q, h],
                recv_sem=ag_recv.at[d, q, h],
                device_id=(right if d == 0 else left,),
                device_id_type=pl.DeviceIdType.MESH,
            )

        barrier_sem = pltpu.get_barrier_semaphore()
        for nbr in (left, right):
            pl.semaphore_signal(
                barrier_sem, inc=1,
                device_id=(nbr,), device_id_type=pl.DeviceIdType.MESH,
            )
        pl.semaphore_wait(barrier_sem, 2)

        gemm(my)
        pending = {}
        for d, q in streams:
            stage[d, q, 0] = out_ref[rows(my), col(d, q)].astype(jnp.bfloat16)
        for d, q in streams:
            rdma = rs_rdma(d, q, 0, stage.at[d, q, 0])
            rdma.start()
            pending[(d, q)] = rdma

        gemm(lax.rem(my + 1, N_DEV))
        gemm(lax.rem(my + (N_DEV - 1), N_DEV))
        gemm_sched = {
            0: [lax.rem(my + 2, N_DEV), lax.rem(my + (N_DEV - 2), N_DEV)],
            1: [lax.rem(my + 3, N_DEV), lax.rem(my + (N_DEV - 3), N_DEV)],
            2: [lax.rem(my + 4, N_DEV)],
        }

        for h in range(N_HOP):
            for c in gemm_sched.get(h, []):
                gemm(c)
            for d, q in streams:
                pending[(d, q)].wait()
                rc = chunk_id(d, h)
                y = rsbuf[d, q, h].astype(jnp.float32) + out_ref[rows(rc), col(d, q)]
                if h < N_HOP - 1:
                    slot = (h + 1) % 2
                    stage[d, q, slot] = y.astype(jnp.bfloat16)
                    rdma = rs_rdma(d, q, h + 1, stage.at[d, q, slot])
                    rdma.start()
                    pending[(d, q)] = rdma
                else:
                    y = y * jax.nn.sigmoid(y)
                    out_ref[rows(rc), col(d, q)] = y
                    stage[d, q, 0] = y.astype(jnp.bfloat16)

        pl.semaphore_signal(phase_sem, inc=1, device_id=(left,),
                            device_id_type=pl.DeviceIdType.MESH)
        pl.semaphore_signal(phase_sem, inc=1, device_id=(right,),
                            device_id_type=pl.DeviceIdType.MESH)
        pl.semaphore_wait(phase_sem, 2)

        for h in range(N_HOP):
            for d, q in streams:
                src = stage.at[d, q, 0] if h == 0 else rsbuf.at[d, q, h - 1]
                rdma = ag_rdma(d, q, h, src)
                rdma.start()
                pending[(d, q)] = rdma
            if h > 0:
                for d, q in streams:
                    if d == 0:
                        rc = lax.rem(my - (h - 1) + N_DEV, N_DEV)
                    else:
                        rc = lax.rem(my + (h - 1), N_DEV)
                    out_ref[rows(rc), col(d, q)] = rsbuf[d, q, h - 1].astype(jnp.float32)
            for d, q in streams:
                pending[(d, q)].wait()
        for d, q in streams:
            if d == 0:
                rc = lax.rem(my - (N_HOP - 1) + N_DEV, N_DEV)
            else:
                rc = lax.rem(my + (N_HOP - 1), N_DEV)
            out_ref[rows(rc), col(d, q)] = rsbuf[d, q, N_HOP - 1].astype(jnp.float32)

    return pl.pallas_call(
        body,
        out_shape=jax.ShapeDtypeStruct((m, n), jnp.float32),
        in_specs=[
            pl.BlockSpec(memory_space=pltpu.VMEM),
            pl.BlockSpec(memory_space=pltpu.VMEM),
        ],
        out_specs=pl.BlockSpec(memory_space=pltpu.VMEM),
        scratch_shapes=[
            pltpu.VMEM((2, 2, N_HOP, chunk, quarter), jnp.bfloat16),
            pltpu.VMEM((2, 2, 2, chunk, quarter), jnp.bfloat16),
            pltpu.SemaphoreType.DMA((2, 2, N_HOP)),
            pltpu.SemaphoreType.DMA((2, 2, N_HOP)),
            pltpu.SemaphoreType.DMA((2, 2, N_HOP)),
            pltpu.SemaphoreType.DMA((2, 2, N_HOP)),
            pltpu.SemaphoreType.REGULAR,
        ],
        compiler_params=pltpu.CompilerParams(
            collective_id=0,
            vmem_limit_bytes=100 * 1024 * 1024,
        ),
    )(x, w_mat)


# device time: 221258 ns/iter; 2.5377x vs baseline; 1.0415x over previous
import jax
import jax.numpy as jnp
from jax import lax
from jax.experimental import pallas as pl
from jax.experimental.pallas import tpu as pltpu

N_DEV = 8
N_HOP = N_DEV - 1


def kernel(x, w_mat):
    m, k = x.shape
    _, n = w_mat.shape
    chunk = m // N_DEV
    quarter = n // 4

    x = x.astype(jnp.bfloat16)
    w_mat = w_mat.astype(jnp.bfloat16)

    streams = [(0, 0), (0, 1), (1, 0), (1, 1)]

    def col(d, q):
        c0 = (2 * d + q) * quarter
        return slice(c0, c0 + quarter)

    def body(x_ref, w_ref, out_ref, acc, rsbuf, stage, tmp,
             rs_send, rs_recv, ag_send, ag_recv, st_sem, phase_sem):
        my = lax.axis_index("i")
        left = lax.rem(my + (N_DEV - 1), N_DEV)
        right = lax.rem(my + 1, N_DEV)

        def rows(c):
            return pl.ds(c * chunk, chunk)

        def chunk_id(d, h):
            if d == 0:
                return lax.rem(my - h - 1 + 2 * N_DEV, N_DEV)
            return lax.rem(my + h + 1, N_DEV)

        def gemm(c):
            acc[rows(c), :] = jnp.dot(
                x_ref[rows(c), :], w_ref[...],
                preferred_element_type=jnp.float32,
            ).astype(jnp.bfloat16)

        def ring_rdma(send_sems, recv_sems, d, q, h, src):
            return pltpu.make_async_remote_copy(
                src_ref=src,
                dst_ref=rsbuf.at[d, q, h],
                send_sem=send_sems.at[d, q, h],
                recv_sem=recv_sems.at[d, q, h],
                device_id=(right if d == 0 else left,),
                device_id_type=pl.DeviceIdType.MESH,
            )

        def store(d, q, rc, value):
            tmp[2 * d + q] = value
            cp = pltpu.make_async_copy(
                tmp.at[2 * d + q],
                out_ref.at[rows(rc), col(d, q)],
                st_sem.at[2 * d + q],
            )
            cp.start()
            return cp

        barrier_sem = pltpu.get_barrier_semaphore()
        for nbr in (left, right):
            pl.semaphore_signal(
                barrier_sem, inc=1,
                device_id=(nbr,), device_id_type=pl.DeviceIdType.MESH,
            )
        pl.semaphore_wait(barrier_sem, 2)

        gemm(my)
        pending = {}
        for d, q in streams:
            rdma = ring_rdma(rs_send, rs_recv, d, q, 0,
                             acc.at[rows(my), col(d, q)])
            rdma.start()
            pending[(d, q)] = rdma

        gemm(lax.rem(my + 1, N_DEV))
        gemm(lax.rem(my + (N_DEV - 1), N_DEV))
        gemm_sched = {
            0: [lax.rem(my + 2, N_DEV), lax.rem(my + (N_DEV - 2), N_DEV)],
            1: [lax.rem(my + 3, N_DEV), lax.rem(my + (N_DEV - 3), N_DEV)],
            2: [lax.rem(my + 4, N_DEV)],
        }

        stores = {}
        for h in range(N_HOP):
            for c in gemm_sched.get(h, []):
                gemm(c)
            for d, q in streams:
                pending[(d, q)].wait()
                rc = chunk_id(d, h)
                if h < N_HOP - 1:
                    slot = (h + 1) % 2
                    stage[d, q, slot] = rsbuf[d, q, h] + acc[rows(rc), col(d, q)]
                    rdma = ring_rdma(rs_send, rs_recv, d, q, h + 1,
                                     stage.at[d, q, slot])
                    rdma.start()
                    pending[(d, q)] = rdma
                else:
                    y = (rsbuf[d, q, h].astype(jnp.float32)
                         + acc[rows(rc), col(d, q)].astype(jnp.float32))
                    y = y * jax.nn.sigmoid(y)
                    stores[(d, q)] = store(d, q, rc, y)
                    stage[d, q, 0] = y.astype(jnp.bfloat16)

        pl.semaphore_signal(phase_sem, inc=1, device_id=(left,),
                            device_id_type=pl.DeviceIdType.MESH)
        pl.semaphore_signal(phase_sem, inc=1, device_id=(right,),
                            device_id_type=pl.DeviceIdType.MESH)
        pl.semaphore_wait(phase_sem, 2)

        def ag_chunk(d, j):
            if d == 0:
                return lax.rem(my - j + N_DEV, N_DEV)
            return lax.rem(my + j, N_DEV)

        for h in range(N_HOP):
            for d, q in streams:
                src = stage.at[d, q, 0] if h == 0 else rsbuf.at[d, q, h - 1]
                rdma = ring_rdma(ag_send, ag_recv, d, q, h, src)
                rdma.start()
                pending[(d, q)] = rdma
            if h > 0:
                for d, q in streams:
                    stores[(d, q)].wait()
                    stores[(d, q)] = store(
                        d, q, ag_chunk(d, h - 1),
                        rsbuf[d, q, h - 1].astype(jnp.float32),
                    )
            for d, q in streams:
                pending[(d, q)].wait()
        for d, q in streams:
            stores[(d, q)].wait()
            stores[(d, q)] = store(
                d, q, ag_chunk(d, N_HOP - 1),
                rsbuf[d, q, N_HOP - 1].astype(jnp.float32),
            )
        for d, q in streams:
            stores[(d, q)].wait()

    return pl.pallas_call(
        body,
        out_shape=jax.ShapeDtypeStruct((m, n), jnp.float32),
        in_specs=[
            pl.BlockSpec(memory_space=pltpu.VMEM),
            pl.BlockSpec(memory_space=pltpu.VMEM),
        ],
        out_specs=pl.BlockSpec(memory_space=pltpu.MemorySpace.HBM),
        scratch_shapes=[
            pltpu.VMEM((m, n), jnp.bfloat16),
            pltpu.VMEM((2, 2, N_HOP, chunk, quarter), jnp.bfloat16),
            pltpu.VMEM((2, 2, 2, chunk, quarter), jnp.bfloat16),
            pltpu.VMEM((4, chunk, quarter), jnp.float32),
            pltpu.SemaphoreType.DMA((2, 2, N_HOP)),
            pltpu.SemaphoreType.DMA((2, 2, N_HOP)),
            pltpu.SemaphoreType.DMA((2, 2, N_HOP)),
            pltpu.SemaphoreType.DMA((2, 2, N_HOP)),
            pltpu.SemaphoreType.DMA((4,)),
            pltpu.SemaphoreType.REGULAR,
        ],
        compiler_params=pltpu.CompilerParams(
            collective_id=0,
            vmem_limit_bytes=100 * 1024 * 1024,
        ),
    )(x, w_mat)
